# baseline (speedup 1.0000x reference)
import functools

import jax
import jax.numpy as jnp
from jax import lax
from jax.experimental import pallas as pl
from jax.experimental.pallas import tpu as pltpu
from jax.experimental.pallas import tpu_sc as plsc

_INFO = plsc.get_sparse_core_info()
_NC, _NS = _INFO.num_cores, _INFO.num_subcores
_NW = _NC * _NS


def _make_gather(n_rows: int, n_cols: int, v_len: int):
    col_chunk = 128
    assert n_cols % (_NW * col_chunk) == 0
    cols_per_w = n_cols // _NW
    n_passes = cols_per_w // col_chunk
    mesh = plsc.VectorSubcoreMesh(core_axis_name="c", subcore_axis_name="s")

    @functools.partial(
        pl.kernel,
        mesh=mesh,
        out_type=jax.ShapeDtypeStruct((n_rows, n_cols), jnp.float32),
        scratch_types=[
            pltpu.VMEM((n_rows, col_chunk), jnp.int32),
            pltpu.VMEM((n_rows, col_chunk), jnp.float32),
            pltpu.VMEM((n_rows, col_chunk), jnp.int32),
            pltpu.VMEM((n_rows, col_chunk), jnp.float32),
            pltpu.VMEM_SHARED((v_len,), jnp.float32),
            pltpu.SemaphoreType.DMA,
            pltpu.SemaphoreType.DMA,
            pltpu.SemaphoreType.DMA,
        ],
    )
    def gather_kernel(
        idx_hbm, table_hbm, out_hbm,
        idx_v, rows_v, idx_h, rows_h, tbl_s, sem, sem_h, sem_s,
    ):
        sid = lax.axis_index("s")
        wid = sid * _NC + lax.axis_index("c")

        @pl.when(sid == 0)
        def _():
            pltpu.make_async_copy(table_hbm, tbl_s, sem_s).start()

        cbase0 = wid * cols_per_w
        pltpu.sync_copy(idx_hbm.at[:, pl.ds(cbase0, col_chunk)], idx_h)

        def fire_h(r, cy):
            pltpu.make_async_copy(
                table_hbm.at[idx_h.at[r]], rows_h.at[r], sem_h
            ).start()
            return cy

        lax.fori_loop(0, n_rows, fire_h, 0)

        @pl.when(sid == 0)
        def _():
            pltpu.make_async_copy(table_hbm, tbl_s, sem_s).wait()

        plsc.subcore_barrier()

        def do_pass(p, carry):
            cbase = wid * cols_per_w + p * col_chunk
            pltpu.sync_copy(idx_hbm.at[:, pl.ds(cbase, col_chunk)], idx_v)

            def fire(r, cy):
                pltpu.make_async_copy(
                    tbl_s.at[idx_v.at[r]], rows_v.at[r], sem
                ).start()
                return cy

            lax.fori_loop(0, n_rows, fire, 0)

            def drain(r, cy):
                pltpu.make_async_copy(
                    tbl_s.at[idx_v.at[r]], rows_v.at[r], sem
                ).wait()
                return cy

            lax.fori_loop(0, n_rows, drain, 0)
            pltpu.sync_copy(rows_v, out_hbm.at[:, pl.ds(cbase, col_chunk)])
            return carry

        lax.fori_loop(1, n_passes, do_pass, 0)

        def drain_h(r, cy):
            pltpu.make_async_copy(
                table_hbm.at[idx_h.at[r]], rows_h.at[r], sem_h
            ).wait()
            return cy

        lax.fori_loop(0, n_rows, drain_h, 0)
        pltpu.sync_copy(rows_h, out_hbm.at[:, pl.ds(cbase0, col_chunk)])

    return gather_kernel


@jax.jit
def kernel(idx, v):
    n, m = idx.shape
    idx_t = idx.astype(jnp.int32).T
    out_t = _make_gather(m, n, v.shape[0])(idx_t, v)
    return out_t.T


# speedup vs baseline: 1.2022x; 1.2022x over previous
import functools

import jax
import jax.numpy as jnp
from jax import lax
from jax.experimental import pallas as pl
from jax.experimental.pallas import tpu as pltpu
from jax.experimental.pallas import tpu_sc as plsc

_INFO = plsc.get_sparse_core_info()
_NC, _NS = _INFO.num_cores, _INFO.num_subcores
_NW = _NC * _NS


def _make_gather(n_rows: int, n_cols: int, v_len: int):
    col_chunk = 128
    assert n_cols % (_NW * col_chunk) == 0
    cols_per_w = n_cols // _NW
    n_passes = cols_per_w // col_chunk
    mesh = plsc.VectorSubcoreMesh(core_axis_name="c", subcore_axis_name="s")

    @functools.partial(
        pl.kernel,
        mesh=mesh,
        out_type=jax.ShapeDtypeStruct((n_rows, n_cols), jnp.float32),
        scratch_types=[
            pltpu.VMEM((2, n_rows, col_chunk), jnp.int32),
            pltpu.VMEM((n_rows, col_chunk), jnp.float32),
            pltpu.VMEM((64,), jnp.int32),
            pltpu.VMEM((64,), jnp.float32),
            pltpu.VMEM_SHARED((v_len,), jnp.float32),
            pltpu.SemaphoreType.DMA,
            pltpu.SemaphoreType.DMA,
        ],
    )
    def gather_kernel(
        idx_hbm, table_hbm, out_hbm,
        idx_v, rows_v, tail_idx, tail_val, tbl_s, sem, sem_i,
    ):
        sid = lax.axis_index("s")
        wid = sid * _NC + lax.axis_index("c")
        w0 = wid * cols_per_w

        n_blocks = v_len // 128
        tail_off = n_blocks * 128
        tail = v_len - tail_off
        per = n_blocks // _NS
        extra = n_blocks - per * _NS
        off = 0
        for k in range(_NS):
            cnt = (per + (1 if k < extra else 0)) * 128

            @pl.when(sid == k)
            def _(off=off, cnt=cnt):
                pltpu.sync_copy(
                    table_hbm.at[pl.ds(off, cnt)], tbl_s.at[pl.ds(off, cnt)]
                )

            off += cnt

        if tail:

            @pl.when(sid == 0)
            def _():
                for j in range(tail // 16):
                    tail_idx[pl.ds(j * 16, 16)] = (
                        lax.iota(jnp.int32, 16) + (tail_off + j * 16)
                    )
                pltpu.async_copy(
                    table_hbm.at[tail_idx], tail_val, sem_i
                ).wait()
                pltpu.async_copy(tail_val, tbl_s.at[tail_idx], sem_i).wait()

        pltpu.make_async_copy(
            idx_hbm.at[:, pl.ds(w0, col_chunk)], idx_v.at[0], sem_i
        ).start()
        plsc.subcore_barrier()

        def do_pass(p, carry):
            buf = lax.rem(p, 2)
            nbuf = lax.rem(p + 1, 2)
            cbase = w0 + p * col_chunk
            pltpu.make_async_copy(
                idx_hbm.at[:, pl.ds(cbase, col_chunk)], idx_v.at[buf], sem_i
            ).wait()

            def fire(r, cy):
                pltpu.make_async_copy(
                    tbl_s.at[idx_v.at[buf, r]], rows_v.at[r], sem
                ).start()
                return cy

            lax.fori_loop(0, n_rows, fire, 0)

            @pl.when(p + 1 < n_passes)
            def _():
                pltpu.make_async_copy(
                    idx_hbm.at[:, pl.ds(cbase + col_chunk, col_chunk)],
                    idx_v.at[nbuf],
                    sem_i,
                ).start()

            def drain(r, cy):
                pltpu.make_async_copy(
                    tbl_s.at[idx_v.at[buf, r]], rows_v.at[r], sem
                ).wait()
                return cy

            lax.fori_loop(0, n_rows, drain, 0)
            pltpu.sync_copy(rows_v, out_hbm.at[:, pl.ds(cbase, col_chunk)])
            return carry

        lax.fori_loop(0, n_passes, do_pass, 0)

    return gather_kernel


@jax.jit
def kernel(idx, v):
    n, m = idx.shape
    idx_t = idx.astype(jnp.int32).T
    out_t = _make_gather(m, n, v.shape[0])(idx_t, v)
    return out_t.T


# speedup vs baseline: 1.2079x; 1.0047x over previous
import functools

import jax
import jax.numpy as jnp
from jax import lax
from jax.experimental import pallas as pl
from jax.experimental.pallas import tpu as pltpu
from jax.experimental.pallas import tpu_sc as plsc

_INFO = plsc.get_sparse_core_info()
_NC, _NS = _INFO.num_cores, _INFO.num_subcores
_NW = _NC * _NS


def _make_gather(n_rows: int, n_cols: int, v_len: int):
    col_chunk = 128
    assert n_cols % (_NW * col_chunk) == 0
    cols_per_w = n_cols // _NW
    n_passes = cols_per_w // col_chunk
    mesh = plsc.VectorSubcoreMesh(core_axis_name="c", subcore_axis_name="s")

    @functools.partial(
        pl.kernel,
        mesh=mesh,
        out_type=jax.ShapeDtypeStruct((n_rows, n_cols), jnp.float32),
        scratch_types=[
            pltpu.VMEM((2, n_rows, col_chunk), jnp.int32),
            pltpu.VMEM((2, n_rows, col_chunk), jnp.float32),
            pltpu.VMEM((64,), jnp.int32),
            pltpu.VMEM((64,), jnp.float32),
            pltpu.VMEM_SHARED((v_len,), jnp.float32),
            pltpu.SemaphoreType.DMA,
            pltpu.SemaphoreType.DMA,
            pltpu.SemaphoreType.DMA,
        ],
    )
    def gather_kernel(
        idx_hbm, table_hbm, out_hbm,
        idx_v, rows_v, tail_idx, tail_val, tbl_s, sem, sem_i, sem_w,
    ):
        sid = lax.axis_index("s")
        wid = sid * _NC + lax.axis_index("c")
        w0 = wid * cols_per_w

        n_blocks = v_len // 128
        tail_off = n_blocks * 128
        tail = v_len - tail_off
        per = n_blocks // _NS
        extra = n_blocks - per * _NS
        off = 0
        for k in range(_NS):
            cnt = (per + (1 if k < extra else 0)) * 128

            @pl.when(sid == k)
            def _(off=off, cnt=cnt):
                pltpu.sync_copy(
                    table_hbm.at[pl.ds(off, cnt)], tbl_s.at[pl.ds(off, cnt)]
                )

            off += cnt

        if tail:

            @pl.when(sid == 0)
            def _():
                for j in range(tail // 16):
                    tail_idx[pl.ds(j * 16, 16)] = (
                        lax.iota(jnp.int32, 16) + (tail_off + j * 16)
                    )
                pltpu.async_copy(
                    table_hbm.at[tail_idx], tail_val, sem_i
                ).wait()
                pltpu.async_copy(tail_val, tbl_s.at[tail_idx], sem_i).wait()

        pltpu.make_async_copy(
            idx_hbm.at[:, pl.ds(w0, col_chunk)], idx_v.at[0], sem_i
        ).start()
        plsc.subcore_barrier()

        out_w0 = out_hbm.at[:, pl.ds(w0, col_chunk)]

        def do_pass(p, carry):
            buf = lax.rem(p, 2)
            nbuf = lax.rem(p + 1, 2)
            cbase = w0 + p * col_chunk
            pltpu.make_async_copy(
                idx_hbm.at[:, pl.ds(cbase, col_chunk)], idx_v.at[buf], sem_i
            ).wait()

            @pl.when(p >= 2)
            def _():
                pltpu.make_async_copy(rows_v.at[buf], out_w0, sem_w).wait()

            def fire(r, cy):
                pltpu.make_async_copy(
                    tbl_s.at[idx_v.at[buf, r]], rows_v.at[buf, r], sem
                ).start()
                return cy

            lax.fori_loop(0, n_rows, fire, 0)

            @pl.when(p + 1 < n_passes)
            def _():
                pltpu.make_async_copy(
                    idx_hbm.at[:, pl.ds(cbase + col_chunk, col_chunk)],
                    idx_v.at[nbuf],
                    sem_i,
                ).start()

            pltpu.make_async_copy(out_w0, rows_v.at[buf], sem).wait()
            pltpu.make_async_copy(
                rows_v.at[buf], out_hbm.at[:, pl.ds(cbase, col_chunk)], sem_w
            ).start()
            return carry

        lax.fori_loop(0, n_passes, do_pass, 0)
        for _ in range(min(2, n_passes)):
            pltpu.make_async_copy(rows_v.at[0], out_w0, sem_w).wait()

    return gather_kernel


@jax.jit
def kernel(idx, v):
    n, m = idx.shape
    idx_t = idx.astype(jnp.int32).T
    out_t = _make_gather(m, n, v.shape[0])(idx_t, v)
    return out_t.T


# speedup vs baseline: 1.2582x; 1.0416x over previous
import functools

import jax
import jax.numpy as jnp
from jax import lax
from jax.experimental import pallas as pl
from jax.experimental.pallas import tpu as pltpu
from jax.experimental.pallas import tpu_sc as plsc

_INFO = plsc.get_sparse_core_info()
_NC, _NS = _INFO.num_cores, _INFO.num_subcores
_NW = _NC * _NS


def _make_gather(n_rows: int, n_cols: int, v_len: int):
    col_chunk = 128
    assert n_cols % (_NW * col_chunk) == 0
    cols_per_w = n_cols // _NW
    n_passes = cols_per_w // col_chunk
    mesh = plsc.VectorSubcoreMesh(core_axis_name="c", subcore_axis_name="s")

    @functools.partial(
        pl.kernel,
        mesh=mesh,
        out_type=jax.ShapeDtypeStruct((n_rows, n_cols), jnp.float32),
        scratch_types=[
            pltpu.VMEM((2, n_rows, col_chunk), jnp.int32),
            pltpu.VMEM((2, n_rows, col_chunk), jnp.float32),
            pltpu.VMEM((64,), jnp.int32),
            pltpu.VMEM((64,), jnp.float32),
            pltpu.VMEM_SHARED((v_len,), jnp.float32),
            pltpu.SemaphoreType.DMA,
            pltpu.SemaphoreType.DMA,
            pltpu.SemaphoreType.DMA,
        ],
    )
    def gather_kernel(
        idx_hbm, table_hbm, out_hbm,
        idx_v, rows_v, tail_idx, tail_val, tbl_s, sem, sem_i, sem_w,
    ):
        sid = lax.axis_index("s")
        wid = sid * _NC + lax.axis_index("c")
        w0 = wid * cols_per_w

        n_blocks = v_len // 128
        tail_off = n_blocks * 128
        tail = v_len - tail_off
        per = n_blocks // _NS
        extra = n_blocks - per * _NS
        off = 0
        for k in range(_NS):
            cnt = (per + (1 if k < extra else 0)) * 128

            @pl.when(sid == k)
            def _(off=off, cnt=cnt):
                pltpu.sync_copy(
                    table_hbm.at[pl.ds(off, cnt)], tbl_s.at[pl.ds(off, cnt)]
                )

            off += cnt

        if tail:

            @pl.when(sid == 0)
            def _():
                for j in range(tail // 16):
                    tail_idx[pl.ds(j * 16, 16)] = (
                        lax.iota(jnp.int32, 16) + (tail_off + j * 16)
                    )
                pltpu.async_copy(
                    table_hbm.at[tail_idx], tail_val, sem_i
                ).wait()
                pltpu.async_copy(tail_val, tbl_s.at[tail_idx], sem_i).wait()

        pltpu.make_async_copy(
            idx_hbm.at[:, pl.ds(w0, col_chunk)], idx_v.at[0], sem_i
        ).start()
        plsc.subcore_barrier()

        out_w0 = out_hbm.at[:, pl.ds(w0, col_chunk)]

        def do_pass(p, carry):
            buf = lax.rem(p, 2)
            nbuf = lax.rem(p + 1, 2)
            cbase = w0 + p * col_chunk
            pltpu.make_async_copy(
                idx_hbm.at[:, pl.ds(cbase, col_chunk)], idx_v.at[buf], sem_i
            ).wait()

            @pl.when(p >= 2)
            def _():
                pltpu.make_async_copy(rows_v.at[buf], out_w0, sem_w).wait()

            @pl.when(p + 1 < n_passes)
            def _():
                pltpu.make_async_copy(
                    idx_hbm.at[:, pl.ds(cbase + col_chunk, col_chunk)],
                    idx_v.at[nbuf],
                    sem_i,
                ).start()

            def fire(r, cy):
                pltpu.make_async_copy(
                    tbl_s.at[idx_v.at[buf, r]], rows_v.at[buf, r], sem
                ).start()
                return cy

            lax.fori_loop(0, n_rows, fire, 0)

            pltpu.make_async_copy(out_w0, rows_v.at[buf], sem).wait()
            pltpu.make_async_copy(
                rows_v.at[buf], out_hbm.at[:, pl.ds(cbase, col_chunk)], sem_w
            ).start()
            return carry

        lax.fori_loop(0, n_passes, do_pass, 0)
        for _ in range(min(2, n_passes)):
            pltpu.make_async_copy(rows_v.at[0], out_w0, sem_w).wait()

    return gather_kernel


@jax.jit
def kernel(idx, v):
    n, m = idx.shape
    idx_t = idx.astype(jnp.int32).T
    out_t = _make_gather(m, n, v.shape[0])(idx_t, v)
    return out_t.T
